# Initial kernel scaffold; baseline (speedup 1.0000x reference)
import jax
import jax.numpy as jnp
from jax.experimental import pallas as pl


def kernel(x_user, x_news, x_source, follows_src, follows_dst, posted_by_src, posted_by_dst, posts_src, posts_dst, publishes_src, publishes_dst, published_by_src, published_by_dst, conv1_follows_W, conv1_follows_b, conv1_posted_by_W, conv1_posted_by_b, conv1_posts_W, conv1_posts_b, conv1_publishes_W, conv1_publishes_b, conv1_published_by_W, conv1_published_by_b, conv2_follows_W, conv2_follows_b, conv2_posted_by_W, conv2_posted_by_b, conv2_posts_W, conv2_posts_b, conv2_publishes_W, conv2_publishes_b, conv2_published_by_W, conv2_published_by_b, lin_user_W, lin_user_b, lin_news_W, lin_news_b, lin_source_W, lin_source_b):
    raise NotImplementedError("write your pallas kernel here")


# speedup vs baseline: 1.1181x; 1.1181x over previous
import functools
import jax
import jax.numpy as jnp
from jax import lax
from jax.experimental import pallas as pl
from jax.experimental.pallas import tpu as pltpu
from jax.experimental.pallas import tpu_sc as plsc

_NU, _NN, _NS = 100000, 50000, 10000
_F = 128

_RELS = [
    ("follows", "user", "user", 160000),
    ("posted_by", "news", "user", 160000),
    ("posts", "user", "news", 160000),
    ("publishes", "source", "news", 60000),
    ("published_by", "news", "source", 60000),
]
_NNODES = {"user": _NU, "news": _NN, "source": _NS}
_REL_BY_NAME = {r[0]: r for r in _RELS}

_EB = 128
_AB = 32
_K = 8
_STRIPE = _AB * _K * 16
_ZB = 2000

_AGG_PLAN = [
    ("user", 13184, 4),
    ("news", 13184, 2),
    ("source", 6016, 1),
]


def _sc_degrees_kernel():
    jobs = []
    for j, (name, st, dt, ne) in enumerate(_RELS):
        jobs.append((2 * j, _NNODES[st], ne))
        jobs.append((2 * j + 1, _NNODES[dt], ne))

    mesh = plsc.VectorSubcoreMesh(core_axis_name="c", subcore_axis_name="s")
    out_type = [jax.ShapeDtypeStruct((n,), jnp.float32) for (_, n, _) in jobs]
    scratch = (
        [pltpu.VMEM_SHARED((n,), jnp.float32) for (_, n, _) in jobs]
        + [
            pltpu.VMEM((_EB,), jnp.int32),
            pltpu.VMEM((96,), jnp.int32),
            pltpu.VMEM((_EB,), jnp.float32),
            pltpu.VMEM((96,), jnp.float32),
            pltpu.VMEM((_ZB,), jnp.float32),
            pltpu.VMEM((_ZB,), jnp.float32),
        ]
    )

    @functools.partial(
        pl.kernel, out_type=out_type, mesh=mesh, scratch_types=scratch,
        name="sc_degrees",
    )
    def k(*refs):
        n_idx = 2 * len(_RELS)
        idx_hbms = refs[:n_idx]
        zeros_hbm = refs[n_idx]
        ones_hbm = refs[n_idx + 1]
        outs = refs[n_idx + 2: n_idx + 2 + len(jobs)]
        sp = refs[n_idx + 2 + len(jobs): n_idx + 2 + 2 * len(jobs)]
        (idx_v, idx_t, ones_v, ones_t, bounce,
         zeros_v) = refs[n_idx + 2 + 2 * len(jobs):]

        c = lax.axis_index("c")
        t = lax.axis_index("s")
        pltpu.sync_copy(ones_hbm.at[pl.ds(0, _EB)], ones_v)
        pltpu.sync_copy(ones_hbm.at[pl.ds(0, 96)], ones_t)
        pltpu.sync_copy(zeros_hbm.at[pl.ds(0, _ZB)], zeros_v)

        for jid, (ai, n, ne) in enumerate(jobs):
            core = jid % 2
            idx_hbm = idx_hbms[ai]
            spj = sp[jid]
            outj = outs[jid]

            @pl.when(c == core)
            def _job(n=n, ne=ne, idx_hbm=idx_hbm, spj=spj, outj=outj):
                nz = n // _ZB

                def zbody(i, carry):
                    off = (i * 16 + t) * _ZB

                    @pl.when(off < n)
                    def _():
                        pltpu.sync_copy(zeros_v, spj.at[pl.ds(off, _ZB)])
                    return carry
                lax.fori_loop(0, (nz + 15) // 16, zbody, 0)
                plsc.subcore_barrier()

                nb = (ne + _EB * 16 - 1) // (_EB * 16)

                def ebody(i, carry):
                    ebase = (i * 16 + t) * _EB

                    @pl.when(ebase + _EB <= ne)
                    def _():
                        pltpu.sync_copy(idx_hbm.at[pl.ds(ebase, _EB)], idx_v)
                        pltpu.sync_copy(ones_v, spj.at[idx_v], add=True)
                    return carry
                lax.fori_loop(0, nb, ebody, 0)
                tail = ne % _EB
                if tail:
                    assert tail == 96

                    @pl.when(t == 0)
                    def _():
                        pltpu.sync_copy(idx_hbm.at[pl.ds(ne - tail, tail)],
                                        idx_t)
                        pltpu.sync_copy(ones_t, spj.at[idx_t], add=True)
                plsc.subcore_barrier()

                def fbody(i, carry):
                    off = (i * 16 + t) * _ZB

                    @pl.when(off < n)
                    def _():
                        pltpu.sync_copy(spj.at[pl.ds(off, _ZB)], bounce)
                        pltpu.sync_copy(bounce, outj.at[pl.ds(off, _ZB)])
                    return carry
                lax.fori_loop(0, (nz + 15) // 16, fbody, 0)
                plsc.subcore_barrier()

    return k


def _degrees(edge_idx):
    zeros = jnp.zeros((_ZB,), jnp.float32)
    ones = jnp.ones((_EB,), jnp.float32)
    return _sc_degrees_kernel()(*edge_idx, zeros, ones)


def _sc_agg_kernel(n_dst, R, npass, rel_plan):
    nrel = len(rel_plan)
    BUF = R + 128
    assert BUF % 256 == 0

    mesh = plsc.VectorSubcoreMesh(core_axis_name="c", subcore_axis_name="s")
    out_type = [jax.ShapeDtypeStruct((n_dst, _F), jnp.float32)
                for _ in range(nrel)]
    scratch = [
        pltpu.VMEM_SHARED((BUF, _F), jnp.float32),
        pltpu.VMEM((_K, _AB), jnp.int32),
        pltpu.VMEM((_K, _AB), jnp.int32),
        pltpu.VMEM((4, _AB, _F), jnp.float32),
        pltpu.VMEM((16, _F), jnp.float32),
        pltpu.VMEM((8, _F), jnp.float32),
    ] + [pltpu.SemaphoreType.DMA] * 8

    @functools.partial(
        pl.kernel, out_type=out_type, mesh=mesh, scratch_types=scratch,
        name=f"sc_agg_{n_dst}")
    def k(*refs):
        ins = refs[: 3 * nrel + 1]
        zeros_hbm = ins[3 * nrel]
        outs = refs[3 * nrel + 1: 3 * nrel + 1 + nrel]
        sc = refs[3 * nrel + 1 + nrel:]
        acc_sp, idx_s2, idx_d2, rows, zeros_v, bounce8 = sc[:6]
        sems = sc[6:14]
        gsem, ssem = sems[:4], sems[4:]

        c = lax.axis_index("c")
        t = lax.axis_index("s")
        pltpu.sync_copy(zeros_hbm, zeros_v)

        zchunks = (BUF // 16) // 16
        fchunks = R // _AB
        base_t = t * (BUF // 16)

        for r in range(nrel):
            ne_pad, _ = rel_plan[r]
            src2, dst2, y_hbm = ins[3 * r: 3 * r + 3]
            n_iters = ne_pad // _STRIPE
            for p in range(npass):
                lo = (2 * p + c) * R
                hi = lo + R

                @pl.when(lo < n_dst)
                def _seg(lo=lo, hi=hi, src2=src2, dst2=dst2, y_hbm=y_hbm,
                         outj=outs[r], n_iters=n_iters):
                    def zbody(j, carry):
                        pltpu.sync_copy(
                            zeros_v,
                            acc_sp.at[pl.ds(base_t + j * 16, 16), :])
                        return carry
                    lax.fori_loop(0, zchunks, zbody, 0)
                    plsc.subcore_barrier()

                    def ebody(i, carry):
                        srow = (i * 16 + t) * _K
                        pltpu.sync_copy(src2.at[pl.ds(srow, _K), :], idx_s2)
                        pltpu.sync_copy(dst2.at[pl.ds(srow, _K), :], idx_d2)

                        for kk in range(_K):
                            def cbody(j, carry2, kk=kk):
                                v = idx_d2[kk, pl.ds(j * 16, 16)]
                                ok = (v >= lo) & (v < hi)
                                loc = jnp.where(ok, v - lo,
                                                R + (v & 127))
                                idx_d2[kk, pl.ds(j * 16, 16)] = loc
                                return carry2
                            lax.fori_loop(0, _AB // 16, cbody, 0)

                        gd = [None] * _K
                        sd = [None] * _K
                        for kk in range(_K):
                            if kk >= 4:
                                sd[kk - 4].wait()
                            gd[kk] = pltpu.async_copy(
                                y_hbm.at[idx_s2.at[kk]], rows.at[kk % 4],
                                gsem[kk % 4])
                            if kk >= 1:
                                gd[kk - 1].wait()
                                sd[kk - 1] = pltpu.async_copy(
                                    rows.at[(kk - 1) % 4],
                                    acc_sp.at[idx_d2.at[kk - 1]],
                                    ssem[(kk - 1) % 4], add=True)
                        gd[_K - 1].wait()
                        sd[_K - 1] = pltpu.async_copy(
                            rows.at[(_K - 1) % 4],
                            acc_sp.at[idx_d2.at[_K - 1]],
                            ssem[(_K - 1) % 4], add=True)
                        for kk in range(_K - 4, _K):
                            sd[kk].wait()
                        return carry
                    lax.fori_loop(0, n_iters, ebody, 0)
                    plsc.subcore_barrier()

                    vfull = jnp.minimum(hi, n_dst) - lo

                    def fbody(j, carry):
                        off = (j * 16 + t) * _AB

                        @pl.when(off + _AB <= vfull)
                        def _():
                            pltpu.sync_copy(acc_sp.at[pl.ds(off, _AB), :],
                                            rows.at[j % 4])
                            pltpu.sync_copy(rows.at[j % 4],
                                            outj.at[pl.ds(lo + off, _AB), :])

                        @pl.when((off < vfull) & (off + _AB > vfull))
                        def _():
                            def sbody(m, carry2):
                                off2 = off + m * 8

                                @pl.when(off2 + 8 <= vfull)
                                def _():
                                    pltpu.sync_copy(
                                        acc_sp.at[pl.ds(off2, 8), :], bounce8)
                                    pltpu.sync_copy(
                                        bounce8,
                                        outj.at[pl.ds(lo + off2, 8), :])
                                return carry2
                            lax.fori_loop(0, _AB // 8, sbody, 0)
                        return carry
                    lax.fori_loop(0, (fchunks + 15) // 16, fbody, 0)
                    plsc.subcore_barrier()

    return k


def _tc_xw(x, deg, W, bn=1000):
    n = x.shape[0]

    def body(x_ref, d_ref, w_ref, o_ref):
        nrm = lax.rsqrt(jnp.maximum(d_ref[...], 1.0))
        o_ref[...] = jnp.dot(x_ref[...] * nrm, w_ref[...],
                             preferred_element_type=jnp.float32)

    return pl.pallas_call(
        body,
        grid=(n // bn,),
        in_specs=[
            pl.BlockSpec((bn, x.shape[1]), lambda i: (i, 0)),
            pl.BlockSpec((bn, 1), lambda i: (i, 0)),
            pl.BlockSpec(W.shape, lambda i: (0, 0)),
        ],
        out_specs=pl.BlockSpec((bn, W.shape[1]), lambda i: (i, 0)),
        out_shape=jax.ShapeDtypeStruct((n, W.shape[1]), jnp.float32),
    )(x, deg, W)


def _tc_combine(accs, degs, biases, bn=1000):
    nrel = len(accs)
    n = accs[0].shape[0]

    def body(*refs):
        acc_refs = refs[:nrel]
        deg_refs = refs[nrel:2 * nrel]
        b_refs = refs[2 * nrel:3 * nrel]
        o_ref = refs[3 * nrel]
        tot = None
        for r in range(nrel):
            nrm = lax.rsqrt(jnp.maximum(deg_refs[r][...], 1.0))
            term = acc_refs[r][...] * nrm + b_refs[r][...]
            tot = term if tot is None else tot + term
        tot = tot * (1.0 / nrel)
        o_ref[...] = jnp.where(tot >= 0, tot, 0.01 * tot)

    in_specs = (
        [pl.BlockSpec((bn, _F), lambda i: (i, 0)) for _ in range(nrel)]
        + [pl.BlockSpec((bn, 1), lambda i: (i, 0)) for _ in range(nrel)]
        + [pl.BlockSpec((1, _F), lambda i: (0, 0)) for _ in range(nrel)]
    )
    return pl.pallas_call(
        body,
        grid=(n // bn,),
        in_specs=in_specs,
        out_specs=pl.BlockSpec((bn, _F), lambda i: (i, 0)),
        out_shape=jax.ShapeDtypeStruct((n, _F), jnp.float32),
    )(*accs, *degs, *[b.reshape(1, -1) for b in biases])


def _tc_final(h, W, b, bn=1000):
    n = h.shape[0]

    def body(h_ref, w_ref, b_ref, o_ref):
        o_ref[...] = jnp.dot(h_ref[...], w_ref[...],
                             preferred_element_type=jnp.float32) + b_ref[...]

    return pl.pallas_call(
        body,
        grid=(n // bn,),
        in_specs=[
            pl.BlockSpec((bn, _F), lambda i: (i, 0)),
            pl.BlockSpec(W.shape, lambda i: (0, 0)),
            pl.BlockSpec((1, W.shape[1]), lambda i: (0, 0)),
        ],
        out_specs=pl.BlockSpec((bn, W.shape[1]), lambda i: (i, 0)),
        out_shape=jax.ShapeDtypeStruct((n, W.shape[1]), jnp.float32),
    )(h, W, b.reshape(1, -1))


_DST_RELS = {
    "user": ["follows", "posted_by"],
    "news": ["posts", "publishes"],
    "source": ["published_by"],
}


def _pad_edges(src, dst, n_src, n_dst):
    ne = src.shape[0]
    ne_pad = ((ne + _STRIPE - 1) // _STRIPE) * _STRIPE
    pad = ne_pad - ne
    if pad:
        ar = jnp.arange(pad, dtype=jnp.int32)
        src = jnp.concatenate([src, (ar * 97) % n_src])
        dst = jnp.concatenate([dst, n_dst + (ar & 63)])
    return src.reshape(ne_pad // _AB, _AB), dst.reshape(ne_pad // _AB, _AB)


def kernel(x_user, x_news, x_source, follows_src, follows_dst, posted_by_src, posted_by_dst, posts_src, posts_dst, publishes_src, publishes_dst, published_by_src, published_by_dst, conv1_follows_W, conv1_follows_b, conv1_posted_by_W, conv1_posted_by_b, conv1_posts_W, conv1_posts_b, conv1_publishes_W, conv1_publishes_b, conv1_published_by_W, conv1_published_by_b, conv2_follows_W, conv2_follows_b, conv2_posted_by_W, conv2_posted_by_b, conv2_posts_W, conv2_posts_b, conv2_publishes_W, conv2_publishes_b, conv2_published_by_W, conv2_published_by_b, lin_user_W, lin_user_b, lin_news_W, lin_news_b, lin_source_W, lin_source_b):
    d = dict(locals())
    edge_idx = []
    for name, st, dt, ne in _RELS:
        edge_idx.append(d[f"{name}_src"])
        edge_idx.append(d[f"{name}_dst"])
    degs = _degrees(edge_idx)
    deg_out = {}
    deg_in = {}
    for j, (name, st, dt, ne) in enumerate(_RELS):
        deg_out[name] = degs[2 * j].reshape(-1, 1)
        deg_in[name] = degs[2 * j + 1].reshape(-1, 1)

    padded = {}
    for name, st, dt, ne in _RELS:
        padded[name] = _pad_edges(d[f"{name}_src"], d[f"{name}_dst"],
                                  _NNODES[st], _NNODES[dt])

    xs = {"user": x_user, "news": x_news, "source": x_source}
    zeros_in = jnp.zeros((16, _F), jnp.float32)

    def hetero(xs_in, layer):
        ys = {}
        for name, st, dt, ne in _RELS:
            ys[name] = _tc_xw(xs_in[st], deg_out[name],
                              d[f"{layer}_{name}_W"])
        out = {}
        for dt_name, R, npass in _AGG_PLAN:
            n_dst = _NNODES[dt_name]
            rels = _DST_RELS[dt_name]
            rel_plan = []
            agg_in = []
            for rn in rels:
                s2, d2 = padded[rn]
                rel_plan.append((s2.shape[0] * _AB, _REL_BY_NAME[rn][3]))
                agg_in += [s2, d2, ys[rn]]
            agg_in.append(zeros_in)
            accs = _sc_agg_kernel(n_dst, R, npass, rel_plan)(*agg_in)
            if not isinstance(accs, (list, tuple)):
                accs = [accs]
            out[dt_name] = _tc_combine(
                list(accs),
                [deg_in[rn] for rn in rels],
                [d[f"{layer}_{rn}_b"] for rn in rels])
        return out

    h = hetero(xs, "conv1")
    h1 = hetero(h, "conv2")
    hout = {
        "user": _tc_final(h1["user"], lin_user_W, lin_user_b),
        "news": _tc_final(h1["news"], lin_news_W, lin_news_b),
        "source": _tc_final(h1["source"], lin_source_W, lin_source_b),
    }
    return (hout["user"], hout["news"], hout["source"],
            h1["user"], h1["news"], h1["source"])


# speedup vs baseline: 1.3782x; 1.2326x over previous
import functools
import jax
import jax.numpy as jnp
from jax import lax
from jax.experimental import pallas as pl
from jax.experimental.pallas import tpu as pltpu
from jax.experimental.pallas import tpu_sc as plsc

_NU, _NN, _NS = 100000, 50000, 10000
_F = 128

_RELS = [
    ("follows", "user", "user", 160000),
    ("posted_by", "news", "user", 160000),
    ("posts", "user", "news", 160000),
    ("publishes", "source", "news", 60000),
    ("published_by", "news", "source", 60000),
]
_NNODES = {"user": _NU, "news": _NN, "source": _NS}
_REL_BY_NAME = {r[0]: r for r in _RELS}

_EB = 128
_AB = 32
_K = 8
_STRIPE = _AB * _K * 16
_ZB = 2000

_AGG_PLAN = [
    ("user", 13184, 4),
    ("news", 13184, 2),
    ("source", 6016, 1),
]


def _sc_degrees_kernel():
    jobs = []
    for j, (name, st, dt, ne) in enumerate(_RELS):
        jobs.append((2 * j, _NNODES[st], ne))
        jobs.append((2 * j + 1, _NNODES[dt], ne))

    mesh = plsc.VectorSubcoreMesh(core_axis_name="c", subcore_axis_name="s")
    out_type = [jax.ShapeDtypeStruct((n,), jnp.float32) for (_, n, _) in jobs]
    scratch = (
        [pltpu.VMEM_SHARED((n,), jnp.float32) for (_, n, _) in jobs]
        + [
            pltpu.VMEM((_EB,), jnp.int32),
            pltpu.VMEM((96,), jnp.int32),
            pltpu.VMEM((_EB,), jnp.float32),
            pltpu.VMEM((96,), jnp.float32),
            pltpu.VMEM((_ZB,), jnp.float32),
            pltpu.VMEM((_ZB,), jnp.float32),
        ]
    )

    @functools.partial(
        pl.kernel, out_type=out_type, mesh=mesh, scratch_types=scratch,
        name="sc_degrees",
    )
    def k(*refs):
        n_idx = 2 * len(_RELS)
        idx_hbms = refs[:n_idx]
        zeros_hbm = refs[n_idx]
        ones_hbm = refs[n_idx + 1]
        outs = refs[n_idx + 2: n_idx + 2 + len(jobs)]
        sp = refs[n_idx + 2 + len(jobs): n_idx + 2 + 2 * len(jobs)]
        (idx_v, idx_t, ones_v, ones_t, bounce,
         zeros_v) = refs[n_idx + 2 + 2 * len(jobs):]

        c = lax.axis_index("c")
        t = lax.axis_index("s")
        pltpu.sync_copy(ones_hbm.at[pl.ds(0, _EB)], ones_v)
        pltpu.sync_copy(ones_hbm.at[pl.ds(0, 96)], ones_t)
        pltpu.sync_copy(zeros_hbm.at[pl.ds(0, _ZB)], zeros_v)

        for jid, (ai, n, ne) in enumerate(jobs):
            core = jid % 2
            idx_hbm = idx_hbms[ai]
            spj = sp[jid]
            outj = outs[jid]

            @pl.when(c == core)
            def _job(n=n, ne=ne, idx_hbm=idx_hbm, spj=spj, outj=outj):
                nz = n // _ZB

                def zbody(i, carry):
                    off = (i * 16 + t) * _ZB

                    @pl.when(off < n)
                    def _():
                        pltpu.sync_copy(zeros_v, spj.at[pl.ds(off, _ZB)])
                    return carry
                lax.fori_loop(0, (nz + 15) // 16, zbody, 0)
                plsc.subcore_barrier()

                nb = (ne + _EB * 16 - 1) // (_EB * 16)

                def ebody(i, carry):
                    ebase = (i * 16 + t) * _EB

                    @pl.when(ebase + _EB <= ne)
                    def _():
                        pltpu.sync_copy(idx_hbm.at[pl.ds(ebase, _EB)], idx_v)
                        pltpu.sync_copy(ones_v, spj.at[idx_v], add=True)
                    return carry
                lax.fori_loop(0, nb, ebody, 0)
                tail = ne % _EB
                if tail:
                    assert tail == 96

                    @pl.when(t == 0)
                    def _():
                        pltpu.sync_copy(idx_hbm.at[pl.ds(ne - tail, tail)],
                                        idx_t)
                        pltpu.sync_copy(ones_t, spj.at[idx_t], add=True)
                plsc.subcore_barrier()

                def fbody(i, carry):
                    off = (i * 16 + t) * _ZB

                    @pl.when(off < n)
                    def _():
                        pltpu.sync_copy(spj.at[pl.ds(off, _ZB)], bounce)
                        pltpu.sync_copy(bounce, outj.at[pl.ds(off, _ZB)])
                    return carry
                lax.fori_loop(0, (nz + 15) // 16, fbody, 0)
                plsc.subcore_barrier()

    return k


def _degrees(edge_idx):
    zeros = jnp.zeros((_ZB,), jnp.float32)
    ones = jnp.ones((_EB,), jnp.float32)
    return _sc_degrees_kernel()(*edge_idx, zeros, ones)


def _sc_agg_kernel(n_dst, R, npass, rel_plan):
    nrel = len(rel_plan)
    BUF = R + 128
    assert BUF % 256 == 0

    mesh = plsc.VectorSubcoreMesh(core_axis_name="c", subcore_axis_name="s")
    out_type = [jax.ShapeDtypeStruct((n_dst, _F), jnp.float32)
                for _ in range(nrel)]
    scratch = [
        pltpu.VMEM_SHARED((BUF, _F), jnp.float32),
        pltpu.VMEM((2, _K, _AB), jnp.int32),
        pltpu.VMEM((2, _K, _AB), jnp.int32),
        pltpu.VMEM((4, _AB, _F), jnp.float32),
        pltpu.VMEM((16, _F), jnp.float32),
        pltpu.VMEM((8, _F), jnp.float32),
    ] + [pltpu.SemaphoreType.DMA] * 10

    @functools.partial(
        pl.kernel, out_type=out_type, mesh=mesh, scratch_types=scratch,
        name=f"sc_agg_{n_dst}")
    def k(*refs):
        ins = refs[: 3 * nrel + 1]
        zeros_hbm = ins[3 * nrel]
        outs = refs[3 * nrel + 1: 3 * nrel + 1 + nrel]
        sc = refs[3 * nrel + 1 + nrel:]
        acc_sp, idx_s2, idx_d2, rows, zeros_v, bounce8 = sc[:6]
        sems = sc[6:16]
        gsem, ssem, isem = sems[:4], sems[4:8], sems[8:]

        c = lax.axis_index("c")
        t = lax.axis_index("s")
        pltpu.sync_copy(zeros_hbm, zeros_v)

        zchunks = (BUF // 16) // 16
        fchunks = R // _AB
        base_t = t * (BUF // 16)

        for r in range(nrel):
            ne_pad, _ = rel_plan[r]
            src2, dst2, y_hbm = ins[3 * r: 3 * r + 3]
            n_iters = ne_pad // _STRIPE
            for p in range(npass):
                lo = (2 * p + c) * R
                hi = lo + R

                @pl.when(lo < n_dst)
                def _seg(lo=lo, hi=hi, src2=src2, dst2=dst2, y_hbm=y_hbm,
                         outj=outs[r], n_iters=n_iters):
                    def zbody(j, carry):
                        pltpu.async_copy(
                            zeros_v,
                            acc_sp.at[pl.ds(base_t + j * 16, 16), :],
                            gsem[0])
                        return carry
                    lax.fori_loop(0, zchunks, zbody, 0)

                    def zwait(j, carry):
                        pltpu.make_async_copy(
                            zeros_v, acc_sp.at[pl.ds(base_t, 16), :],
                            gsem[0]).wait()
                        return carry
                    lax.fori_loop(0, zchunks, zwait, 0)
                    plsc.subcore_barrier()

                    def _fetch_idx(i, b, sync):
                        srow = (i * 16 + t) * _K
                        if sync:
                            pltpu.sync_copy(src2.at[pl.ds(srow, _K), :],
                                            idx_s2.at[b])
                            pltpu.sync_copy(dst2.at[pl.ds(srow, _K), :],
                                            idx_d2.at[b])
                        else:
                            pltpu.async_copy(src2.at[pl.ds(srow, _K), :],
                                             idx_s2.at[b], isem[b])
                            pltpu.async_copy(dst2.at[pl.ds(srow, _K), :],
                                             idx_d2.at[b], isem[b])

                    def _wait_idx(b):
                        pltpu.make_async_copy(
                            src2.at[pl.ds(0, _K), :], idx_s2.at[b],
                            isem[b]).wait()
                        pltpu.make_async_copy(
                            src2.at[pl.ds(0, _K), :], idx_d2.at[b],
                            isem[b]).wait()

                    _fetch_idx(0, 0, True)

                    def ebody(i2, carry):
                        for m in range(2):
                            i = i2 * 2 + m

                            @pl.when(i < n_iters)
                            def _(i=i, m=m):
                                @pl.when(i + 1 < n_iters)
                                def _():
                                    _fetch_idx(i + 1, 1 - m, False)

                                @pl.when(i > 0)
                                def _():
                                    _wait_idx(m)

                                for kk in range(_K):
                                    def cbody(j, carry2, kk=kk):
                                        v = idx_d2[m, kk, pl.ds(j * 16, 16)]
                                        ok = (v >= lo) & (v < hi)
                                        loc = jnp.where(ok, v - lo,
                                                        R + (v & 127))
                                        idx_d2[m, kk, pl.ds(j * 16, 16)] = loc
                                        return carry2
                                    lax.fori_loop(0, _AB // 16, cbody, 0)

                                gd = [None] * _K
                                sd = [None] * _K
                                for kk in range(_K):
                                    if kk >= 4:
                                        sd[kk - 4].wait()
                                    gd[kk] = pltpu.async_copy(
                                        y_hbm.at[idx_s2.at[m, kk]],
                                        rows.at[kk % 4], gsem[kk % 4])
                                    if kk >= 1:
                                        gd[kk - 1].wait()
                                        sd[kk - 1] = pltpu.async_copy(
                                            rows.at[(kk - 1) % 4],
                                            acc_sp.at[idx_d2.at[m, kk - 1]],
                                            ssem[(kk - 1) % 4], add=True)
                                gd[_K - 1].wait()
                                sd[_K - 1] = pltpu.async_copy(
                                    rows.at[(_K - 1) % 4],
                                    acc_sp.at[idx_d2.at[m, _K - 1]],
                                    ssem[(_K - 1) % 4], add=True)
                                for kk in range(_K - 4, _K):
                                    sd[kk].wait()
                        return carry
                    lax.fori_loop(0, (n_iters + 1) // 2, ebody, 0)
                    plsc.subcore_barrier()

                    vfull = jnp.minimum(hi, n_dst) - lo
                    nfl = (fchunks + 15) // 16
                    nfl4 = (nfl + 3) // 4

                    def _full(j):
                        return (((j * 16 + t) * _AB + _AB <= vfull)
                                & (j >= 0) & (j < nfl))

                    def _wait_write(j, slot):
                        @pl.when(_full(j))
                        def _():
                            pltpu.make_async_copy(
                                rows.at[slot], outj.at[pl.ds(lo, _AB), :],
                                ssem[slot]).wait()

                    def fbody(j4, carry):
                        for m in range(4):
                            j = j4 * 4 + m
                            off = (j * 16 + t) * _AB
                            _wait_write(j - 4, m)

                            @pl.when(_full(j))
                            def _(off=off, m=m):
                                pltpu.sync_copy(
                                    acc_sp.at[pl.ds(off, _AB), :],
                                    rows.at[m])
                                pltpu.async_copy(
                                    rows.at[m],
                                    outj.at[pl.ds(lo + off, _AB), :],
                                    ssem[m])

                            @pl.when((off < vfull) & (off + _AB > vfull))
                            def _(off=off):
                                def sbody(mm, carry2):
                                    off2 = off + mm * 8

                                    @pl.when(off2 + 8 <= vfull)
                                    def _():
                                        pltpu.sync_copy(
                                            acc_sp.at[pl.ds(off2, 8), :],
                                            bounce8)
                                        pltpu.sync_copy(
                                            bounce8,
                                            outj.at[pl.ds(lo + off2, 8), :])
                                    return carry2
                                lax.fori_loop(0, _AB // 8, sbody, 0)
                        return carry
                    lax.fori_loop(0, nfl4, fbody, 0)
                    for m in range(4):
                        _wait_write((nfl4 - 1) * 4 + m, m)
                    plsc.subcore_barrier()

    return k


def _tc_xw(x, deg, W, bn=1000):
    n = x.shape[0]

    def body(x_ref, d_ref, w_ref, o_ref):
        nrm = lax.rsqrt(jnp.maximum(d_ref[...], 1.0))
        o_ref[...] = jnp.dot(x_ref[...] * nrm, w_ref[...],
                             preferred_element_type=jnp.float32)

    return pl.pallas_call(
        body,
        grid=(n // bn,),
        in_specs=[
            pl.BlockSpec((bn, x.shape[1]), lambda i: (i, 0)),
            pl.BlockSpec((bn, 1), lambda i: (i, 0)),
            pl.BlockSpec(W.shape, lambda i: (0, 0)),
        ],
        out_specs=pl.BlockSpec((bn, W.shape[1]), lambda i: (i, 0)),
        out_shape=jax.ShapeDtypeStruct((n, W.shape[1]), jnp.float32),
    )(x, deg, W)


def _tc_combine(accs, degs, biases, bn=1000):
    nrel = len(accs)
    n = accs[0].shape[0]

    def body(*refs):
        acc_refs = refs[:nrel]
        deg_refs = refs[nrel:2 * nrel]
        b_refs = refs[2 * nrel:3 * nrel]
        o_ref = refs[3 * nrel]
        tot = None
        for r in range(nrel):
            nrm = lax.rsqrt(jnp.maximum(deg_refs[r][...], 1.0))
            term = acc_refs[r][...] * nrm + b_refs[r][...]
            tot = term if tot is None else tot + term
        tot = tot * (1.0 / nrel)
        o_ref[...] = jnp.where(tot >= 0, tot, 0.01 * tot)

    in_specs = (
        [pl.BlockSpec((bn, _F), lambda i: (i, 0)) for _ in range(nrel)]
        + [pl.BlockSpec((bn, 1), lambda i: (i, 0)) for _ in range(nrel)]
        + [pl.BlockSpec((1, _F), lambda i: (0, 0)) for _ in range(nrel)]
    )
    return pl.pallas_call(
        body,
        grid=(n // bn,),
        in_specs=in_specs,
        out_specs=pl.BlockSpec((bn, _F), lambda i: (i, 0)),
        out_shape=jax.ShapeDtypeStruct((n, _F), jnp.float32),
    )(*accs, *degs, *[b.reshape(1, -1) for b in biases])


def _tc_final(h, W, b, bn=1000):
    n = h.shape[0]

    def body(h_ref, w_ref, b_ref, o_ref):
        o_ref[...] = jnp.dot(h_ref[...], w_ref[...],
                             preferred_element_type=jnp.float32) + b_ref[...]

    return pl.pallas_call(
        body,
        grid=(n // bn,),
        in_specs=[
            pl.BlockSpec((bn, _F), lambda i: (i, 0)),
            pl.BlockSpec(W.shape, lambda i: (0, 0)),
            pl.BlockSpec((1, W.shape[1]), lambda i: (0, 0)),
        ],
        out_specs=pl.BlockSpec((bn, W.shape[1]), lambda i: (i, 0)),
        out_shape=jax.ShapeDtypeStruct((n, W.shape[1]), jnp.float32),
    )(h, W, b.reshape(1, -1))


_DST_RELS = {
    "user": ["follows", "posted_by"],
    "news": ["posts", "publishes"],
    "source": ["published_by"],
}


def _pad_edges(src, dst, n_src, n_dst):
    ne = src.shape[0]
    ne_pad = ((ne + _STRIPE - 1) // _STRIPE) * _STRIPE
    pad = ne_pad - ne
    if pad:
        ar = jnp.arange(pad, dtype=jnp.int32)
        src = jnp.concatenate([src, (ar * 97) % n_src])
        dst = jnp.concatenate([dst, n_dst + (ar & 63)])
    return src.reshape(ne_pad // _AB, _AB), dst.reshape(ne_pad // _AB, _AB)


def kernel(x_user, x_news, x_source, follows_src, follows_dst, posted_by_src, posted_by_dst, posts_src, posts_dst, publishes_src, publishes_dst, published_by_src, published_by_dst, conv1_follows_W, conv1_follows_b, conv1_posted_by_W, conv1_posted_by_b, conv1_posts_W, conv1_posts_b, conv1_publishes_W, conv1_publishes_b, conv1_published_by_W, conv1_published_by_b, conv2_follows_W, conv2_follows_b, conv2_posted_by_W, conv2_posted_by_b, conv2_posts_W, conv2_posts_b, conv2_publishes_W, conv2_publishes_b, conv2_published_by_W, conv2_published_by_b, lin_user_W, lin_user_b, lin_news_W, lin_news_b, lin_source_W, lin_source_b):
    d = dict(locals())
    edge_idx = []
    for name, st, dt, ne in _RELS:
        edge_idx.append(d[f"{name}_src"])
        edge_idx.append(d[f"{name}_dst"])
    degs = _degrees(edge_idx)
    deg_out = {}
    deg_in = {}
    for j, (name, st, dt, ne) in enumerate(_RELS):
        deg_out[name] = degs[2 * j].reshape(-1, 1)
        deg_in[name] = degs[2 * j + 1].reshape(-1, 1)

    padded = {}
    for name, st, dt, ne in _RELS:
        padded[name] = _pad_edges(d[f"{name}_src"], d[f"{name}_dst"],
                                  _NNODES[st], _NNODES[dt])

    xs = {"user": x_user, "news": x_news, "source": x_source}
    zeros_in = jnp.zeros((16, _F), jnp.float32)

    def hetero(xs_in, layer):
        ys = {}
        for name, st, dt, ne in _RELS:
            ys[name] = _tc_xw(xs_in[st], deg_out[name],
                              d[f"{layer}_{name}_W"])
        out = {}
        for dt_name, R, npass in _AGG_PLAN:
            n_dst = _NNODES[dt_name]
            rels = _DST_RELS[dt_name]
            rel_plan = []
            agg_in = []
            for rn in rels:
                s2, d2 = padded[rn]
                rel_plan.append((s2.shape[0] * _AB, _REL_BY_NAME[rn][3]))
                agg_in += [s2, d2, ys[rn]]
            agg_in.append(zeros_in)
            accs = _sc_agg_kernel(n_dst, R, npass, rel_plan)(*agg_in)
            if not isinstance(accs, (list, tuple)):
                accs = [accs]
            out[dt_name] = _tc_combine(
                list(accs),
                [deg_in[rn] for rn in rels],
                [d[f"{layer}_{rn}_b"] for rn in rels])
        return out

    h = hetero(xs, "conv1")
    h1 = hetero(h, "conv2")
    hout = {
        "user": _tc_final(h1["user"], lin_user_W, lin_user_b),
        "news": _tc_final(h1["news"], lin_news_W, lin_news_b),
        "source": _tc_final(h1["source"], lin_source_W, lin_source_b),
    }
    return (hout["user"], hout["news"], hout["source"],
            h1["user"], h1["news"], h1["source"])


# speedup vs baseline: 1.3860x; 1.0057x over previous
import functools
import jax
import jax.numpy as jnp
from jax import lax
from jax.experimental import pallas as pl
from jax.experimental.pallas import tpu as pltpu
from jax.experimental.pallas import tpu_sc as plsc

_NU, _NN, _NS = 100000, 50000, 10000
_F = 128

_RELS = [
    ("follows", "user", "user", 160000),
    ("posted_by", "news", "user", 160000),
    ("posts", "user", "news", 160000),
    ("publishes", "source", "news", 60000),
    ("published_by", "news", "source", 60000),
]
_NNODES = {"user": _NU, "news": _NN, "source": _NS}
_REL_BY_NAME = {r[0]: r for r in _RELS}

_EB = 128
_AB = 32
_K = 8
_STRIPE = _AB * _K * 16
_ZB = 2000

_AGG_PLAN = [
    ("user", 13184, 4),
    ("news", 13184, 2),
    ("source", 6016, 1),
]


def _sc_degrees_kernel():
    jobs = []
    for j, (name, st, dt, ne) in enumerate(_RELS):
        jobs.append((2 * j, _NNODES[st], ne))
        jobs.append((2 * j + 1, _NNODES[dt], ne))

    mesh = plsc.VectorSubcoreMesh(core_axis_name="c", subcore_axis_name="s")
    out_type = [jax.ShapeDtypeStruct((n,), jnp.float32) for (_, n, _) in jobs]
    scratch = (
        [pltpu.VMEM_SHARED((n,), jnp.float32) for (_, n, _) in jobs]
        + [
            pltpu.VMEM((_EB,), jnp.int32),
            pltpu.VMEM((96,), jnp.int32),
            pltpu.VMEM((_EB,), jnp.float32),
            pltpu.VMEM((96,), jnp.float32),
            pltpu.VMEM((_ZB,), jnp.float32),
            pltpu.VMEM((_ZB,), jnp.float32),
        ]
    )

    @functools.partial(
        pl.kernel, out_type=out_type, mesh=mesh, scratch_types=scratch,
        name="sc_degrees",
    )
    def k(*refs):
        n_idx = 2 * len(_RELS)
        idx_hbms = refs[:n_idx]
        zeros_hbm = refs[n_idx]
        ones_hbm = refs[n_idx + 1]
        outs = refs[n_idx + 2: n_idx + 2 + len(jobs)]
        sp = refs[n_idx + 2 + len(jobs): n_idx + 2 + 2 * len(jobs)]
        (idx_v, idx_t, ones_v, ones_t, bounce,
         zeros_v) = refs[n_idx + 2 + 2 * len(jobs):]

        c = lax.axis_index("c")
        t = lax.axis_index("s")
        pltpu.sync_copy(ones_hbm.at[pl.ds(0, _EB)], ones_v)
        pltpu.sync_copy(ones_hbm.at[pl.ds(0, 96)], ones_t)
        pltpu.sync_copy(zeros_hbm.at[pl.ds(0, _ZB)], zeros_v)

        for jid, (ai, n, ne) in enumerate(jobs):
            core = jid % 2
            idx_hbm = idx_hbms[ai]
            spj = sp[jid]
            outj = outs[jid]

            @pl.when(c == core)
            def _job(n=n, ne=ne, idx_hbm=idx_hbm, spj=spj, outj=outj):
                nz = n // _ZB

                def zbody(i, carry):
                    off = (i * 16 + t) * _ZB

                    @pl.when(off < n)
                    def _():
                        pltpu.sync_copy(zeros_v, spj.at[pl.ds(off, _ZB)])
                    return carry
                lax.fori_loop(0, (nz + 15) // 16, zbody, 0)
                plsc.subcore_barrier()

                nb = (ne + _EB * 16 - 1) // (_EB * 16)

                def ebody(i, carry):
                    ebase = (i * 16 + t) * _EB

                    @pl.when(ebase + _EB <= ne)
                    def _():
                        pltpu.sync_copy(idx_hbm.at[pl.ds(ebase, _EB)], idx_v)
                        pltpu.sync_copy(ones_v, spj.at[idx_v], add=True)
                    return carry
                lax.fori_loop(0, nb, ebody, 0)
                tail = ne % _EB
                if tail:
                    assert tail == 96

                    @pl.when(t == 0)
                    def _():
                        pltpu.sync_copy(idx_hbm.at[pl.ds(ne - tail, tail)],
                                        idx_t)
                        pltpu.sync_copy(ones_t, spj.at[idx_t], add=True)
                plsc.subcore_barrier()

                def fbody(i, carry):
                    off = (i * 16 + t) * _ZB

                    @pl.when(off < n)
                    def _():
                        pltpu.sync_copy(spj.at[pl.ds(off, _ZB)], bounce)
                        pltpu.sync_copy(bounce, outj.at[pl.ds(off, _ZB)])
                    return carry
                lax.fori_loop(0, (nz + 15) // 16, fbody, 0)
                plsc.subcore_barrier()

    return k


def _degrees(edge_idx):
    zeros = jnp.zeros((_ZB,), jnp.float32)
    ones = jnp.ones((_EB,), jnp.float32)
    return _sc_degrees_kernel()(*edge_idx, zeros, ones)


def _sc_agg_kernel(n_dst, R, npass, rel_plan):
    nrel = len(rel_plan)
    BUF = R + 128
    assert BUF % 256 == 0

    mesh = plsc.VectorSubcoreMesh(core_axis_name="c", subcore_axis_name="s")
    out_type = [jax.ShapeDtypeStruct((n_dst, _F), jnp.float32)
                for _ in range(nrel)]
    scratch = [
        pltpu.VMEM_SHARED((BUF, _F), jnp.float32),
        pltpu.VMEM((2, _K, _AB), jnp.int32),
        pltpu.VMEM((2, _K, _AB), jnp.int32),
        pltpu.VMEM((4, _AB, _F), jnp.float32),
        pltpu.VMEM((16, _F), jnp.float32),
        pltpu.VMEM((8, _F), jnp.float32),
    ] + [pltpu.SemaphoreType.DMA] * 10

    @functools.partial(
        pl.kernel, out_type=out_type, mesh=mesh, scratch_types=scratch,
        name=f"sc_agg_{n_dst}")
    def k(*refs):
        ins = refs[: 3 * nrel + 1]
        zeros_hbm = ins[3 * nrel]
        outs = refs[3 * nrel + 1: 3 * nrel + 1 + nrel]
        sc = refs[3 * nrel + 1 + nrel:]
        acc_sp, idx_s2, idx_d2, rows, zeros_v, bounce8 = sc[:6]
        sems = sc[6:16]
        gsem, ssem, isem = sems[:4], sems[4:8], sems[8:]

        c = lax.axis_index("c")
        t = lax.axis_index("s")
        pltpu.sync_copy(zeros_hbm, zeros_v)

        zchunks = (BUF // 16) // 16
        fchunks = R // _AB
        base_t = t * (BUF // 16)

        for r in range(nrel):
            ne_pad, _ = rel_plan[r]
            src2, dst2, y_hbm = ins[3 * r: 3 * r + 3]
            n_iters = ne_pad // _STRIPE

            def pbody(p, pcarry, src2=src2, dst2=dst2, y_hbm=y_hbm,
                      outj=outs[r], n_iters=n_iters):
                lo = (2 * p + c) * R
                hi = lo + R

                @pl.when(lo < n_dst)
                def _seg(lo=lo, hi=hi, src2=src2, dst2=dst2, y_hbm=y_hbm,
                         outj=outj, n_iters=n_iters):
                    def zbody(j, carry):
                        pltpu.async_copy(
                            zeros_v,
                            acc_sp.at[pl.ds(base_t + j * 16, 16), :],
                            gsem[0])
                        return carry
                    lax.fori_loop(0, zchunks, zbody, 0)

                    def zwait(j, carry):
                        pltpu.make_async_copy(
                            zeros_v, acc_sp.at[pl.ds(base_t, 16), :],
                            gsem[0]).wait()
                        return carry
                    lax.fori_loop(0, zchunks, zwait, 0)
                    plsc.subcore_barrier()

                    def _fetch_idx(i, b, sync):
                        srow = (i * 16 + t) * _K
                        if sync:
                            pltpu.sync_copy(src2.at[pl.ds(srow, _K), :],
                                            idx_s2.at[b])
                            pltpu.sync_copy(dst2.at[pl.ds(srow, _K), :],
                                            idx_d2.at[b])
                        else:
                            pltpu.async_copy(src2.at[pl.ds(srow, _K), :],
                                             idx_s2.at[b], isem[b])
                            pltpu.async_copy(dst2.at[pl.ds(srow, _K), :],
                                             idx_d2.at[b], isem[b])

                    def _wait_idx(b):
                        pltpu.make_async_copy(
                            src2.at[pl.ds(0, _K), :], idx_s2.at[b],
                            isem[b]).wait()
                        pltpu.make_async_copy(
                            src2.at[pl.ds(0, _K), :], idx_d2.at[b],
                            isem[b]).wait()

                    _fetch_idx(0, 0, True)

                    def ebody(i2, carry):
                        for m in range(2):
                            i = i2 * 2 + m

                            @pl.when(i < n_iters)
                            def _(i=i, m=m):
                                @pl.when(i + 1 < n_iters)
                                def _():
                                    _fetch_idx(i + 1, 1 - m, False)

                                @pl.when(i > 0)
                                def _():
                                    _wait_idx(m)

                                for kk in range(_K):
                                    def cbody(j, carry2, kk=kk):
                                        v = idx_d2[m, kk, pl.ds(j * 16, 16)]
                                        ok = (v >= lo) & (v < hi)
                                        loc = jnp.where(ok, v - lo,
                                                        R + (v & 127))
                                        idx_d2[m, kk, pl.ds(j * 16, 16)] = loc
                                        return carry2
                                    lax.fori_loop(0, _AB // 16, cbody, 0)

                                gd = [None] * _K
                                sd = [None] * _K
                                for kk in range(_K):
                                    if kk >= 4:
                                        sd[kk - 4].wait()
                                    gd[kk] = pltpu.async_copy(
                                        y_hbm.at[idx_s2.at[m, kk]],
                                        rows.at[kk % 4], gsem[kk % 4])
                                    if kk >= 1:
                                        gd[kk - 1].wait()
                                        sd[kk - 1] = pltpu.async_copy(
                                            rows.at[(kk - 1) % 4],
                                            acc_sp.at[idx_d2.at[m, kk - 1]],
                                            ssem[(kk - 1) % 4], add=True)
                                gd[_K - 1].wait()
                                sd[_K - 1] = pltpu.async_copy(
                                    rows.at[(_K - 1) % 4],
                                    acc_sp.at[idx_d2.at[m, _K - 1]],
                                    ssem[(_K - 1) % 4], add=True)
                                for kk in range(_K - 4, _K):
                                    sd[kk].wait()
                        return carry
                    lax.fori_loop(0, (n_iters + 1) // 2, ebody, 0)
                    plsc.subcore_barrier()

                    vfull = jnp.minimum(hi, n_dst) - lo
                    nfl = (fchunks + 15) // 16
                    nfl4 = (nfl + 3) // 4

                    def _full(j):
                        return (((j * 16 + t) * _AB + _AB <= vfull)
                                & (j >= 0) & (j < nfl))

                    def _wait_write(j, slot):
                        @pl.when(_full(j))
                        def _():
                            pltpu.make_async_copy(
                                rows.at[slot], outj.at[pl.ds(lo, _AB), :],
                                ssem[slot]).wait()

                    def fbody(j4, carry):
                        for m in range(4):
                            j = j4 * 4 + m
                            off = (j * 16 + t) * _AB
                            _wait_write(j - 4, m)

                            @pl.when(_full(j))
                            def _(off=off, m=m):
                                pltpu.sync_copy(
                                    acc_sp.at[pl.ds(off, _AB), :],
                                    rows.at[m])
                                pltpu.async_copy(
                                    rows.at[m],
                                    outj.at[pl.ds(lo + off, _AB), :],
                                    ssem[m])

                            @pl.when((off < vfull) & (off + _AB > vfull))
                            def _(off=off):
                                def sbody(mm, carry2):
                                    off2 = off + mm * 8

                                    @pl.when(off2 + 8 <= vfull)
                                    def _():
                                        pltpu.sync_copy(
                                            acc_sp.at[pl.ds(off2, 8), :],
                                            bounce8)
                                        pltpu.sync_copy(
                                            bounce8,
                                            outj.at[pl.ds(lo + off2, 8), :])
                                    return carry2
                                lax.fori_loop(0, _AB // 8, sbody, 0)
                        return carry
                    lax.fori_loop(0, nfl4, fbody, 0)
                    for m in range(4):
                        _wait_write((nfl4 - 1) * 4 + m, m)
                    plsc.subcore_barrier()
                return pcarry
            lax.fori_loop(0, npass, pbody, 0)

    return k


def _tc_xw(x, deg, W, bn=1000):
    n = x.shape[0]

    def body(x_ref, d_ref, w_ref, o_ref):
        nrm = lax.rsqrt(jnp.maximum(d_ref[...], 1.0))
        o_ref[...] = jnp.dot(x_ref[...] * nrm, w_ref[...],
                             preferred_element_type=jnp.float32)

    return pl.pallas_call(
        body,
        grid=(n // bn,),
        in_specs=[
            pl.BlockSpec((bn, x.shape[1]), lambda i: (i, 0)),
            pl.BlockSpec((bn, 1), lambda i: (i, 0)),
            pl.BlockSpec(W.shape, lambda i: (0, 0)),
        ],
        out_specs=pl.BlockSpec((bn, W.shape[1]), lambda i: (i, 0)),
        out_shape=jax.ShapeDtypeStruct((n, W.shape[1]), jnp.float32),
    )(x, deg, W)


def _tc_combine(accs, degs, biases, bn=1000):
    nrel = len(accs)
    n = accs[0].shape[0]

    def body(*refs):
        acc_refs = refs[:nrel]
        deg_refs = refs[nrel:2 * nrel]
        b_refs = refs[2 * nrel:3 * nrel]
        o_ref = refs[3 * nrel]
        tot = None
        for r in range(nrel):
            nrm = lax.rsqrt(jnp.maximum(deg_refs[r][...], 1.0))
            term = acc_refs[r][...] * nrm + b_refs[r][...]
            tot = term if tot is None else tot + term
        tot = tot * (1.0 / nrel)
        o_ref[...] = jnp.where(tot >= 0, tot, 0.01 * tot)

    in_specs = (
        [pl.BlockSpec((bn, _F), lambda i: (i, 0)) for _ in range(nrel)]
        + [pl.BlockSpec((bn, 1), lambda i: (i, 0)) for _ in range(nrel)]
        + [pl.BlockSpec((1, _F), lambda i: (0, 0)) for _ in range(nrel)]
    )
    return pl.pallas_call(
        body,
        grid=(n // bn,),
        in_specs=in_specs,
        out_specs=pl.BlockSpec((bn, _F), lambda i: (i, 0)),
        out_shape=jax.ShapeDtypeStruct((n, _F), jnp.float32),
    )(*accs, *degs, *[b.reshape(1, -1) for b in biases])


def _tc_final(h, W, b, bn=1000):
    n = h.shape[0]

    def body(h_ref, w_ref, b_ref, o_ref):
        o_ref[...] = jnp.dot(h_ref[...], w_ref[...],
                             preferred_element_type=jnp.float32) + b_ref[...]

    return pl.pallas_call(
        body,
        grid=(n // bn,),
        in_specs=[
            pl.BlockSpec((bn, _F), lambda i: (i, 0)),
            pl.BlockSpec(W.shape, lambda i: (0, 0)),
            pl.BlockSpec((1, W.shape[1]), lambda i: (0, 0)),
        ],
        out_specs=pl.BlockSpec((bn, W.shape[1]), lambda i: (i, 0)),
        out_shape=jax.ShapeDtypeStruct((n, W.shape[1]), jnp.float32),
    )(h, W, b.reshape(1, -1))


_DST_RELS = {
    "user": ["follows", "posted_by"],
    "news": ["posts", "publishes"],
    "source": ["published_by"],
}


def _pad_edges(src, dst, n_src, n_dst):
    ne = src.shape[0]
    ne_pad = ((ne + _STRIPE - 1) // _STRIPE) * _STRIPE
    pad = ne_pad - ne
    if pad:
        ar = jnp.arange(pad, dtype=jnp.int32)
        src = jnp.concatenate([src, (ar * 97) % n_src])
        dst = jnp.concatenate([dst, n_dst + (ar & 63)])
    return src.reshape(ne_pad // _AB, _AB), dst.reshape(ne_pad // _AB, _AB)


def kernel(x_user, x_news, x_source, follows_src, follows_dst, posted_by_src, posted_by_dst, posts_src, posts_dst, publishes_src, publishes_dst, published_by_src, published_by_dst, conv1_follows_W, conv1_follows_b, conv1_posted_by_W, conv1_posted_by_b, conv1_posts_W, conv1_posts_b, conv1_publishes_W, conv1_publishes_b, conv1_published_by_W, conv1_published_by_b, conv2_follows_W, conv2_follows_b, conv2_posted_by_W, conv2_posted_by_b, conv2_posts_W, conv2_posts_b, conv2_publishes_W, conv2_publishes_b, conv2_published_by_W, conv2_published_by_b, lin_user_W, lin_user_b, lin_news_W, lin_news_b, lin_source_W, lin_source_b):
    d = dict(locals())
    edge_idx = []
    for name, st, dt, ne in _RELS:
        edge_idx.append(d[f"{name}_src"])
        edge_idx.append(d[f"{name}_dst"])
    degs = _degrees(edge_idx)
    deg_out = {}
    deg_in = {}
    for j, (name, st, dt, ne) in enumerate(_RELS):
        deg_out[name] = degs[2 * j].reshape(-1, 1)
        deg_in[name] = degs[2 * j + 1].reshape(-1, 1)

    padded = {}
    for name, st, dt, ne in _RELS:
        padded[name] = _pad_edges(d[f"{name}_src"], d[f"{name}_dst"],
                                  _NNODES[st], _NNODES[dt])

    xs = {"user": x_user, "news": x_news, "source": x_source}
    zeros_in = jnp.zeros((16, _F), jnp.float32)

    def hetero(xs_in, layer):
        ys = {}
        for name, st, dt, ne in _RELS:
            ys[name] = _tc_xw(xs_in[st], deg_out[name],
                              d[f"{layer}_{name}_W"])
        out = {}
        for dt_name, R, npass in _AGG_PLAN:
            n_dst = _NNODES[dt_name]
            rels = _DST_RELS[dt_name]
            rel_plan = []
            agg_in = []
            for rn in rels:
                s2, d2 = padded[rn]
                rel_plan.append((s2.shape[0] * _AB, _REL_BY_NAME[rn][3]))
                agg_in += [s2, d2, ys[rn]]
            agg_in.append(zeros_in)
            accs = _sc_agg_kernel(n_dst, R, npass, rel_plan)(*agg_in)
            if not isinstance(accs, (list, tuple)):
                accs = [accs]
            out[dt_name] = _tc_combine(
                list(accs),
                [deg_in[rn] for rn in rels],
                [d[f"{layer}_{rn}_b"] for rn in rels])
        return out

    h = hetero(xs, "conv1")
    h1 = hetero(h, "conv2")
    hout = {
        "user": _tc_final(h1["user"], lin_user_W, lin_user_b),
        "news": _tc_final(h1["news"], lin_news_W, lin_news_b),
        "source": _tc_final(h1["source"], lin_source_W, lin_source_b),
    }
    return (hout["user"], hout["news"], hout["source"],
            h1["user"], h1["news"], h1["source"])


# speedup vs baseline: 1.3911x; 1.0036x over previous
import functools
import jax
import jax.numpy as jnp
from jax import lax
from jax.experimental import pallas as pl
from jax.experimental.pallas import tpu as pltpu
from jax.experimental.pallas import tpu_sc as plsc

_NU, _NN, _NS = 100000, 50000, 10000
_F = 128

_RELS = [
    ("follows", "user", "user", 160000),
    ("posted_by", "news", "user", 160000),
    ("posts", "user", "news", 160000),
    ("publishes", "source", "news", 60000),
    ("published_by", "news", "source", 60000),
]
_NNODES = {"user": _NU, "news": _NN, "source": _NS}
_REL_BY_NAME = {r[0]: r for r in _RELS}

_EB = 128
_AB = 32
_K = 8
_STRIPE = _AB * _K * 16
_ZB = 2000

_AGG_PLAN = [
    ("user", 13184, 4),
    ("news", 13184, 2),
    ("source", 6016, 1),
]


def _sc_degrees_kernel():
    jobs = []
    for j, (name, st, dt, ne) in enumerate(_RELS):
        jobs.append((2 * j, _NNODES[st], ne))
        jobs.append((2 * j + 1, _NNODES[dt], ne))

    mesh = plsc.VectorSubcoreMesh(core_axis_name="c", subcore_axis_name="s")
    out_type = [jax.ShapeDtypeStruct((n,), jnp.float32) for (_, n, _) in jobs]
    scratch = (
        [pltpu.VMEM_SHARED((n,), jnp.float32) for (_, n, _) in jobs]
        + [
            pltpu.VMEM((4, _EB), jnp.int32),
            pltpu.VMEM((96,), jnp.int32),
            pltpu.VMEM((_EB,), jnp.float32),
            pltpu.VMEM((96,), jnp.float32),
            pltpu.VMEM((_ZB,), jnp.float32),
            pltpu.VMEM((_ZB,), jnp.float32),
        ]
        + [pltpu.SemaphoreType.DMA] * 8
    )

    @functools.partial(
        pl.kernel, out_type=out_type, mesh=mesh, scratch_types=scratch,
        name="sc_degrees",
    )
    def k(*refs):
        n_idx = 2 * len(_RELS)
        idx_hbms = refs[:n_idx]
        zeros_hbm = refs[n_idx]
        ones_hbm = refs[n_idx + 1]
        outs = refs[n_idx + 2: n_idx + 2 + len(jobs)]
        sp = refs[n_idx + 2 + len(jobs): n_idx + 2 + 2 * len(jobs)]
        rest = refs[n_idx + 2 + 2 * len(jobs):]
        idx_v, idx_t, ones_v, ones_t, bounce, zeros_v = rest[:6]
        dsem = rest[6:10]
        dssem = rest[10:14]

        c = lax.axis_index("c")
        t = lax.axis_index("s")
        pltpu.sync_copy(ones_hbm.at[pl.ds(0, _EB)], ones_v)
        pltpu.sync_copy(ones_hbm.at[pl.ds(0, 96)], ones_t)
        pltpu.sync_copy(zeros_hbm.at[pl.ds(0, _ZB)], zeros_v)

        for jid, (ai, n, ne) in enumerate(jobs):
            core = jid % 2
            idx_hbm = idx_hbms[ai]
            spj = sp[jid]
            outj = outs[jid]

            @pl.when(c == core)
            def _job(n=n, ne=ne, idx_hbm=idx_hbm, spj=spj, outj=outj):
                nz = n // _ZB

                def zbody(i, carry):
                    off = (i * 16 + t) * _ZB

                    @pl.when(off < n)
                    def _():
                        pltpu.sync_copy(zeros_v, spj.at[pl.ds(off, _ZB)])
                    return carry
                lax.fori_loop(0, (nz + 15) // 16, zbody, 0)
                plsc.subcore_barrier()

                nb = (ne + _EB * 16 - 1) // (_EB * 16)
                nb4 = (nb + 3) // 4

                def _in_range(i):
                    return ((i >= 0) & (i < nb)
                            & ((i * 16 + t) * _EB + _EB <= ne))

                def ebody(i4, carry):
                    for mm in range(4):
                        i = i4 * 4 + mm

                        @pl.when(_in_range(i - 4))
                        def _(mm=mm):
                            pltpu.make_async_copy(
                                ones_v, spj.at[idx_v.at[mm]],
                                dssem[mm]).wait()

                        @pl.when(_in_range(i))
                        def _(i=i, mm=mm):
                            ebase = (i * 16 + t) * _EB
                            pltpu.async_copy(
                                idx_hbm.at[pl.ds(ebase, _EB)],
                                idx_v.at[mm], dsem[mm])
                            pltpu.make_async_copy(
                                idx_hbm.at[pl.ds(0, _EB)], idx_v.at[mm],
                                dsem[mm]).wait()
                            pltpu.async_copy(ones_v, spj.at[idx_v.at[mm]],
                                             dssem[mm], add=True)
                    return carry
                lax.fori_loop(0, nb4, ebody, 0)
                for mm in range(4):
                    @pl.when(_in_range((nb4 - 1) * 4 + mm))
                    def _(mm=mm):
                        pltpu.make_async_copy(
                            ones_v, spj.at[idx_v.at[mm]], dssem[mm]).wait()
                tail = ne % _EB
                if tail:
                    assert tail == 96

                    @pl.when(t == 0)
                    def _():
                        pltpu.sync_copy(idx_hbm.at[pl.ds(ne - tail, tail)],
                                        idx_t)
                        pltpu.sync_copy(ones_t, spj.at[idx_t], add=True)
                plsc.subcore_barrier()

                def fbody(i, carry):
                    off = (i * 16 + t) * _ZB

                    @pl.when(off < n)
                    def _():
                        pltpu.sync_copy(spj.at[pl.ds(off, _ZB)], bounce)
                        pltpu.sync_copy(bounce, outj.at[pl.ds(off, _ZB)])
                    return carry
                lax.fori_loop(0, (nz + 15) // 16, fbody, 0)
                plsc.subcore_barrier()

    return k


def _degrees(edge_idx):
    zeros = jnp.zeros((_ZB,), jnp.float32)
    ones = jnp.ones((_EB,), jnp.float32)
    return _sc_degrees_kernel()(*edge_idx, zeros, ones)


def _sc_agg_kernel(n_dst, R, npass, rel_plan):
    nrel = len(rel_plan)
    BUF = R + 128
    assert BUF % 256 == 0

    mesh = plsc.VectorSubcoreMesh(core_axis_name="c", subcore_axis_name="s")
    out_type = [jax.ShapeDtypeStruct((n_dst, _F), jnp.float32)
                for _ in range(nrel)]
    scratch = [
        pltpu.VMEM_SHARED((BUF, _F), jnp.float32),
        pltpu.VMEM((2, _K, _AB), jnp.int32),
        pltpu.VMEM((2, _K, _AB), jnp.int32),
        pltpu.VMEM((4, _AB, _F), jnp.float32),
        pltpu.VMEM((16, _F), jnp.float32),
        pltpu.VMEM((8, _F), jnp.float32),
    ] + [pltpu.SemaphoreType.DMA] * 10

    @functools.partial(
        pl.kernel, out_type=out_type, mesh=mesh, scratch_types=scratch,
        name=f"sc_agg_{n_dst}")
    def k(*refs):
        ins = refs[: 3 * nrel + 1]
        zeros_hbm = ins[3 * nrel]
        outs = refs[3 * nrel + 1: 3 * nrel + 1 + nrel]
        sc = refs[3 * nrel + 1 + nrel:]
        acc_sp, idx_s2, idx_d2, rows, zeros_v, bounce8 = sc[:6]
        sems = sc[6:16]
        gsem, ssem, isem = sems[:4], sems[4:8], sems[8:]

        c = lax.axis_index("c")
        t = lax.axis_index("s")
        pltpu.sync_copy(zeros_hbm, zeros_v)

        zchunks = (BUF // 16) // 16
        fchunks = R // _AB
        base_t = t * (BUF // 16)

        for r in range(nrel):
            ne_pad, _ = rel_plan[r]
            src2, dst2, y_hbm = ins[3 * r: 3 * r + 3]
            n_iters = ne_pad // _STRIPE

            def pbody(p, pcarry, src2=src2, dst2=dst2, y_hbm=y_hbm,
                      outj=outs[r], n_iters=n_iters):
                lo = (2 * p + c) * R
                hi = lo + R

                @pl.when(lo < n_dst)
                def _seg(lo=lo, hi=hi, src2=src2, dst2=dst2, y_hbm=y_hbm,
                         outj=outj, n_iters=n_iters):
                    def zbody(j, carry):
                        pltpu.async_copy(
                            zeros_v,
                            acc_sp.at[pl.ds(base_t + j * 16, 16), :],
                            gsem[0])
                        return carry
                    lax.fori_loop(0, zchunks, zbody, 0)

                    def zwait(j, carry):
                        pltpu.make_async_copy(
                            zeros_v, acc_sp.at[pl.ds(base_t, 16), :],
                            gsem[0]).wait()
                        return carry
                    lax.fori_loop(0, zchunks, zwait, 0)
                    plsc.subcore_barrier()

                    def _fetch_idx(i, b, sync):
                        srow = (i * 16 + t) * _K
                        if sync:
                            pltpu.sync_copy(src2.at[pl.ds(srow, _K), :],
                                            idx_s2.at[b])
                            pltpu.sync_copy(dst2.at[pl.ds(srow, _K), :],
                                            idx_d2.at[b])
                        else:
                            pltpu.async_copy(src2.at[pl.ds(srow, _K), :],
                                             idx_s2.at[b], isem[b])
                            pltpu.async_copy(dst2.at[pl.ds(srow, _K), :],
                                             idx_d2.at[b], isem[b])

                    def _wait_idx(b):
                        pltpu.make_async_copy(
                            src2.at[pl.ds(0, _K), :], idx_s2.at[b],
                            isem[b]).wait()
                        pltpu.make_async_copy(
                            src2.at[pl.ds(0, _K), :], idx_d2.at[b],
                            isem[b]).wait()

                    _fetch_idx(0, 0, True)

                    def ebody(i2, carry):
                        for m in range(2):
                            i = i2 * 2 + m

                            @pl.when(i < n_iters)
                            def _(i=i, m=m):
                                @pl.when(i + 1 < n_iters)
                                def _():
                                    _fetch_idx(i + 1, 1 - m, False)

                                @pl.when(i > 0)
                                def _():
                                    _wait_idx(m)

                                for kk in range(_K):
                                    def cbody(j, carry2, kk=kk):
                                        v = idx_d2[m, kk, pl.ds(j * 16, 16)]
                                        ok = (v >= lo) & (v < hi)
                                        loc = jnp.where(ok, v - lo,
                                                        R + (v & 127))
                                        idx_d2[m, kk, pl.ds(j * 16, 16)] = loc
                                        return carry2
                                    lax.fori_loop(0, _AB // 16, cbody, 0)

                                gd = [None] * _K
                                sd = [None] * _K
                                for kk in range(_K):
                                    if kk >= 4:
                                        sd[kk - 4].wait()
                                    gd[kk] = pltpu.async_copy(
                                        y_hbm.at[idx_s2.at[m, kk]],
                                        rows.at[kk % 4], gsem[kk % 4])
                                    if kk >= 1:
                                        gd[kk - 1].wait()
                                        sd[kk - 1] = pltpu.async_copy(
                                            rows.at[(kk - 1) % 4],
                                            acc_sp.at[idx_d2.at[m, kk - 1]],
                                            ssem[(kk - 1) % 4], add=True)
                                gd[_K - 1].wait()
                                sd[_K - 1] = pltpu.async_copy(
                                    rows.at[(_K - 1) % 4],
                                    acc_sp.at[idx_d2.at[m, _K - 1]],
                                    ssem[(_K - 1) % 4], add=True)
                                for kk in range(_K - 4, _K):
                                    sd[kk].wait()
                        return carry
                    lax.fori_loop(0, (n_iters + 1) // 2, ebody, 0)
                    plsc.subcore_barrier()

                    vfull = jnp.minimum(hi, n_dst) - lo
                    nfl = (fchunks + 15) // 16
                    nfl4 = (nfl + 3) // 4

                    def _full(j):
                        return (((j * 16 + t) * _AB + _AB <= vfull)
                                & (j >= 0) & (j < nfl))

                    def _wait_write(j, slot):
                        @pl.when(_full(j))
                        def _():
                            pltpu.make_async_copy(
                                rows.at[slot], outj.at[pl.ds(lo, _AB), :],
                                ssem[slot]).wait()

                    def fbody(j4, carry):
                        for m in range(4):
                            j = j4 * 4 + m
                            off = (j * 16 + t) * _AB
                            _wait_write(j - 4, m)

                            @pl.when(_full(j))
                            def _(off=off, m=m):
                                pltpu.sync_copy(
                                    acc_sp.at[pl.ds(off, _AB), :],
                                    rows.at[m])
                                pltpu.async_copy(
                                    rows.at[m],
                                    outj.at[pl.ds(lo + off, _AB), :],
                                    ssem[m])

                            @pl.when((off < vfull) & (off + _AB > vfull))
                            def _(off=off):
                                def sbody(mm, carry2):
                                    off2 = off + mm * 8

                                    @pl.when(off2 + 8 <= vfull)
                                    def _():
                                        pltpu.sync_copy(
                                            acc_sp.at[pl.ds(off2, 8), :],
                                            bounce8)
                                        pltpu.sync_copy(
                                            bounce8,
                                            outj.at[pl.ds(lo + off2, 8), :])
                                    return carry2
                                lax.fori_loop(0, _AB // 8, sbody, 0)
                        return carry
                    lax.fori_loop(0, nfl4, fbody, 0)
                    for m in range(4):
                        _wait_write((nfl4 - 1) * 4 + m, m)
                    plsc.subcore_barrier()
                return pcarry
            lax.fori_loop(0, npass, pbody, 0)

    return k


def _tc_xw(x, deg, W, bn=1000):
    n = x.shape[0]

    def body(x_ref, d_ref, w_ref, o_ref):
        nrm = lax.rsqrt(jnp.maximum(d_ref[...], 1.0))
        o_ref[...] = jnp.dot(x_ref[...] * nrm, w_ref[...],
                             preferred_element_type=jnp.float32)

    return pl.pallas_call(
        body,
        grid=(n // bn,),
        in_specs=[
            pl.BlockSpec((bn, x.shape[1]), lambda i: (i, 0)),
            pl.BlockSpec((bn, 1), lambda i: (i, 0)),
            pl.BlockSpec(W.shape, lambda i: (0, 0)),
        ],
        out_specs=pl.BlockSpec((bn, W.shape[1]), lambda i: (i, 0)),
        out_shape=jax.ShapeDtypeStruct((n, W.shape[1]), jnp.float32),
    )(x, deg, W)


def _tc_combine(accs, degs, biases, bn=1000):
    nrel = len(accs)
    n = accs[0].shape[0]

    def body(*refs):
        acc_refs = refs[:nrel]
        deg_refs = refs[nrel:2 * nrel]
        b_refs = refs[2 * nrel:3 * nrel]
        o_ref = refs[3 * nrel]
        tot = None
        for r in range(nrel):
            nrm = lax.rsqrt(jnp.maximum(deg_refs[r][...], 1.0))
            term = acc_refs[r][...] * nrm + b_refs[r][...]
            tot = term if tot is None else tot + term
        tot = tot * (1.0 / nrel)
        o_ref[...] = jnp.where(tot >= 0, tot, 0.01 * tot)

    in_specs = (
        [pl.BlockSpec((bn, _F), lambda i: (i, 0)) for _ in range(nrel)]
        + [pl.BlockSpec((bn, 1), lambda i: (i, 0)) for _ in range(nrel)]
        + [pl.BlockSpec((1, _F), lambda i: (0, 0)) for _ in range(nrel)]
    )
    return pl.pallas_call(
        body,
        grid=(n // bn,),
        in_specs=in_specs,
        out_specs=pl.BlockSpec((bn, _F), lambda i: (i, 0)),
        out_shape=jax.ShapeDtypeStruct((n, _F), jnp.float32),
    )(*accs, *degs, *[b.reshape(1, -1) for b in biases])


def _tc_final(h, W, b, bn=1000):
    n = h.shape[0]

    def body(h_ref, w_ref, b_ref, o_ref):
        o_ref[...] = jnp.dot(h_ref[...], w_ref[...],
                             preferred_element_type=jnp.float32) + b_ref[...]

    return pl.pallas_call(
        body,
        grid=(n // bn,),
        in_specs=[
            pl.BlockSpec((bn, _F), lambda i: (i, 0)),
            pl.BlockSpec(W.shape, lambda i: (0, 0)),
            pl.BlockSpec((1, W.shape[1]), lambda i: (0, 0)),
        ],
        out_specs=pl.BlockSpec((bn, W.shape[1]), lambda i: (i, 0)),
        out_shape=jax.ShapeDtypeStruct((n, W.shape[1]), jnp.float32),
    )(h, W, b.reshape(1, -1))


_DST_RELS = {
    "user": ["follows", "posted_by"],
    "news": ["posts", "publishes"],
    "source": ["published_by"],
}


def _pad_edges(src, dst, n_src, n_dst):
    ne = src.shape[0]
    ne_pad = ((ne + _STRIPE - 1) // _STRIPE) * _STRIPE
    pad = ne_pad - ne
    if pad:
        ar = jnp.arange(pad, dtype=jnp.int32)
        src = jnp.concatenate([src, (ar * 97) % n_src])
        dst = jnp.concatenate([dst, n_dst + (ar & 63)])
    return src.reshape(ne_pad // _AB, _AB), dst.reshape(ne_pad // _AB, _AB)


def kernel(x_user, x_news, x_source, follows_src, follows_dst, posted_by_src, posted_by_dst, posts_src, posts_dst, publishes_src, publishes_dst, published_by_src, published_by_dst, conv1_follows_W, conv1_follows_b, conv1_posted_by_W, conv1_posted_by_b, conv1_posts_W, conv1_posts_b, conv1_publishes_W, conv1_publishes_b, conv1_published_by_W, conv1_published_by_b, conv2_follows_W, conv2_follows_b, conv2_posted_by_W, conv2_posted_by_b, conv2_posts_W, conv2_posts_b, conv2_publishes_W, conv2_publishes_b, conv2_published_by_W, conv2_published_by_b, lin_user_W, lin_user_b, lin_news_W, lin_news_b, lin_source_W, lin_source_b):
    d = dict(locals())
    edge_idx = []
    for name, st, dt, ne in _RELS:
        edge_idx.append(d[f"{name}_src"])
        edge_idx.append(d[f"{name}_dst"])
    degs = _degrees(edge_idx)
    deg_out = {}
    deg_in = {}
    for j, (name, st, dt, ne) in enumerate(_RELS):
        deg_out[name] = degs[2 * j].reshape(-1, 1)
        deg_in[name] = degs[2 * j + 1].reshape(-1, 1)

    padded = {}
    for name, st, dt, ne in _RELS:
        padded[name] = _pad_edges(d[f"{name}_src"], d[f"{name}_dst"],
                                  _NNODES[st], _NNODES[dt])

    xs = {"user": x_user, "news": x_news, "source": x_source}
    zeros_in = jnp.zeros((16, _F), jnp.float32)

    def hetero(xs_in, layer):
        ys = {}
        for name, st, dt, ne in _RELS:
            ys[name] = _tc_xw(xs_in[st], deg_out[name],
                              d[f"{layer}_{name}_W"])
        out = {}
        for dt_name, R, npass in _AGG_PLAN:
            n_dst = _NNODES[dt_name]
            rels = _DST_RELS[dt_name]
            rel_plan = []
            agg_in = []
            for rn in rels:
                s2, d2 = padded[rn]
                rel_plan.append((s2.shape[0] * _AB, _REL_BY_NAME[rn][3]))
                agg_in += [s2, d2, ys[rn]]
            agg_in.append(zeros_in)
            accs = _sc_agg_kernel(n_dst, R, npass, rel_plan)(*agg_in)
            if not isinstance(accs, (list, tuple)):
                accs = [accs]
            out[dt_name] = _tc_combine(
                list(accs),
                [deg_in[rn] for rn in rels],
                [d[f"{layer}_{rn}_b"] for rn in rels])
        return out

    h = hetero(xs, "conv1")
    h1 = hetero(h, "conv2")
    hout = {
        "user": _tc_final(h1["user"], lin_user_W, lin_user_b),
        "news": _tc_final(h1["news"], lin_news_W, lin_news_b),
        "source": _tc_final(h1["source"], lin_source_W, lin_source_b),
    }
    return (hout["user"], hout["news"], hout["source"],
            h1["user"], h1["news"], h1["source"])


# speedup vs baseline: 1.4307x; 1.0284x over previous
import functools
import jax
import jax.numpy as jnp
from jax import lax
from jax.experimental import pallas as pl
from jax.experimental.pallas import tpu as pltpu
from jax.experimental.pallas import tpu_sc as plsc

_NU, _NN, _NS = 100000, 50000, 10000
_F = 128

_RELS = [
    ("follows", "user", "user", 160000),
    ("posted_by", "news", "user", 160000),
    ("posts", "user", "news", 160000),
    ("publishes", "source", "news", 60000),
    ("published_by", "news", "source", 60000),
]
_NNODES = {"user": _NU, "news": _NN, "source": _NS}
_REL_BY_NAME = {r[0]: r for r in _RELS}

_EB = 128
_AB = 32
_K = 16
_STRIPE = _AB * _K * 16
_ZB = 2000

_AGG_PLAN = [
    ("user", 12672, 4),
    ("news", 12672, 2),
    ("source", 6016, 1),
]


def _sc_degrees_kernel():
    jobs = []
    for j, (name, st, dt, ne) in enumerate(_RELS):
        jobs.append((2 * j, _NNODES[st], ne))
        jobs.append((2 * j + 1, _NNODES[dt], ne))

    mesh = plsc.VectorSubcoreMesh(core_axis_name="c", subcore_axis_name="s")
    out_type = [jax.ShapeDtypeStruct((n,), jnp.float32) for (_, n, _) in jobs]
    scratch = (
        [pltpu.VMEM_SHARED((n,), jnp.float32) for (_, n, _) in jobs]
        + [
            pltpu.VMEM((4, _EB), jnp.int32),
            pltpu.VMEM((96,), jnp.int32),
            pltpu.VMEM((_EB,), jnp.float32),
            pltpu.VMEM((96,), jnp.float32),
            pltpu.VMEM((_ZB,), jnp.float32),
            pltpu.VMEM((_ZB,), jnp.float32),
        ]
        + [pltpu.SemaphoreType.DMA] * 8
    )

    @functools.partial(
        pl.kernel, out_type=out_type, mesh=mesh, scratch_types=scratch,
        name="sc_degrees",
    )
    def k(*refs):
        n_idx = 2 * len(_RELS)
        idx_hbms = refs[:n_idx]
        zeros_hbm = refs[n_idx]
        ones_hbm = refs[n_idx + 1]
        outs = refs[n_idx + 2: n_idx + 2 + len(jobs)]
        sp = refs[n_idx + 2 + len(jobs): n_idx + 2 + 2 * len(jobs)]
        rest = refs[n_idx + 2 + 2 * len(jobs):]
        idx_v, idx_t, ones_v, ones_t, bounce, zeros_v = rest[:6]
        dsem = rest[6:10]
        dssem = rest[10:14]

        c = lax.axis_index("c")
        t = lax.axis_index("s")
        pltpu.sync_copy(ones_hbm.at[pl.ds(0, _EB)], ones_v)
        pltpu.sync_copy(ones_hbm.at[pl.ds(0, 96)], ones_t)
        pltpu.sync_copy(zeros_hbm.at[pl.ds(0, _ZB)], zeros_v)

        for jid, (ai, n, ne) in enumerate(jobs):
            core = jid % 2
            idx_hbm = idx_hbms[ai]
            spj = sp[jid]
            outj = outs[jid]

            @pl.when(c == core)
            def _job(n=n, ne=ne, idx_hbm=idx_hbm, spj=spj, outj=outj):
                nz = n // _ZB

                def zbody(i, carry):
                    off = (i * 16 + t) * _ZB

                    @pl.when(off < n)
                    def _():
                        pltpu.sync_copy(zeros_v, spj.at[pl.ds(off, _ZB)])
                    return carry
                lax.fori_loop(0, (nz + 15) // 16, zbody, 0)
                plsc.subcore_barrier()

                nb = (ne + _EB * 16 - 1) // (_EB * 16)
                nb4 = (nb + 3) // 4

                def _in_range(i):
                    return ((i >= 0) & (i < nb)
                            & ((i * 16 + t) * _EB + _EB <= ne))

                def ebody(i4, carry):
                    for mm in range(4):
                        i = i4 * 4 + mm

                        @pl.when(_in_range(i - 4))
                        def _(mm=mm):
                            pltpu.make_async_copy(
                                ones_v, spj.at[idx_v.at[mm]],
                                dssem[mm]).wait()

                        @pl.when(_in_range(i))
                        def _(i=i, mm=mm):
                            ebase = (i * 16 + t) * _EB
                            pltpu.async_copy(
                                idx_hbm.at[pl.ds(ebase, _EB)],
                                idx_v.at[mm], dsem[mm])
                            pltpu.make_async_copy(
                                idx_hbm.at[pl.ds(0, _EB)], idx_v.at[mm],
                                dsem[mm]).wait()
                            pltpu.async_copy(ones_v, spj.at[idx_v.at[mm]],
                                             dssem[mm], add=True)
                    return carry
                lax.fori_loop(0, nb4, ebody, 0)
                for mm in range(4):
                    @pl.when(_in_range((nb4 - 1) * 4 + mm))
                    def _(mm=mm):
                        pltpu.make_async_copy(
                            ones_v, spj.at[idx_v.at[mm]], dssem[mm]).wait()
                tail = ne % _EB
                if tail:
                    assert tail == 96

                    @pl.when(t == 0)
                    def _():
                        pltpu.sync_copy(idx_hbm.at[pl.ds(ne - tail, tail)],
                                        idx_t)
                        pltpu.sync_copy(ones_t, spj.at[idx_t], add=True)
                plsc.subcore_barrier()

                def fbody(i, carry):
                    off = (i * 16 + t) * _ZB

                    @pl.when(off < n)
                    def _():
                        pltpu.sync_copy(spj.at[pl.ds(off, _ZB)], bounce)
                        pltpu.sync_copy(bounce, outj.at[pl.ds(off, _ZB)])
                    return carry
                lax.fori_loop(0, (nz + 15) // 16, fbody, 0)
                plsc.subcore_barrier()

    return k


def _degrees(edge_idx):
    zeros = jnp.zeros((_ZB,), jnp.float32)
    ones = jnp.ones((_EB,), jnp.float32)
    return _sc_degrees_kernel()(*edge_idx, zeros, ones)


def _sc_agg_kernel(n_dst, R, npass, rel_plan):
    nrel = len(rel_plan)
    BUF = R + 128
    assert BUF % 256 == 0

    mesh = plsc.VectorSubcoreMesh(core_axis_name="c", subcore_axis_name="s")
    out_type = [jax.ShapeDtypeStruct((n_dst, _F), jnp.float32)
                for _ in range(nrel)]
    scratch = [
        pltpu.VMEM_SHARED((BUF, _F), jnp.float32),
        pltpu.VMEM((2, _K, _AB), jnp.int32),
        pltpu.VMEM((2, _K, _AB), jnp.int32),
        pltpu.VMEM((4, _AB, _F), jnp.float32),
        pltpu.VMEM((16, _F), jnp.float32),
        pltpu.VMEM((8, _F), jnp.float32),
    ] + [pltpu.SemaphoreType.DMA] * 10

    @functools.partial(
        pl.kernel, out_type=out_type, mesh=mesh, scratch_types=scratch,
        name=f"sc_agg_{n_dst}")
    def k(*refs):
        ins = refs[: 3 * nrel + 1]
        zeros_hbm = ins[3 * nrel]
        outs = refs[3 * nrel + 1: 3 * nrel + 1 + nrel]
        sc = refs[3 * nrel + 1 + nrel:]
        acc_sp, idx_s2, idx_d2, rows, zeros_v, bounce8 = sc[:6]
        sems = sc[6:16]
        gsem, ssem, isem = sems[:4], sems[4:8], sems[8:]

        c = lax.axis_index("c")
        t = lax.axis_index("s")
        pltpu.sync_copy(zeros_hbm, zeros_v)

        zchunks = (BUF // 16) // 16
        fchunks = R // _AB
        base_t = t * (BUF // 16)

        for r in range(nrel):
            ne_pad, _ = rel_plan[r]
            src2, dst2, y_hbm = ins[3 * r: 3 * r + 3]
            n_iters = ne_pad // _STRIPE

            def pbody(p, pcarry, src2=src2, dst2=dst2, y_hbm=y_hbm,
                      outj=outs[r], n_iters=n_iters):
                lo = (2 * p + c) * R
                hi = lo + R

                @pl.when(lo < n_dst)
                def _seg(lo=lo, hi=hi, src2=src2, dst2=dst2, y_hbm=y_hbm,
                         outj=outj, n_iters=n_iters):
                    def zbody(j, carry):
                        pltpu.async_copy(
                            zeros_v,
                            acc_sp.at[pl.ds(base_t + j * 16, 16), :],
                            gsem[0])
                        return carry
                    lax.fori_loop(0, zchunks, zbody, 0)

                    def zwait(j, carry):
                        pltpu.make_async_copy(
                            zeros_v, acc_sp.at[pl.ds(base_t, 16), :],
                            gsem[0]).wait()
                        return carry
                    lax.fori_loop(0, zchunks, zwait, 0)
                    plsc.subcore_barrier()

                    def _fetch_idx(i, b, sync):
                        srow = (i * 16 + t) * _K
                        if sync:
                            pltpu.sync_copy(src2.at[pl.ds(srow, _K), :],
                                            idx_s2.at[b])
                            pltpu.sync_copy(dst2.at[pl.ds(srow, _K), :],
                                            idx_d2.at[b])
                        else:
                            pltpu.async_copy(src2.at[pl.ds(srow, _K), :],
                                             idx_s2.at[b], isem[b])
                            pltpu.async_copy(dst2.at[pl.ds(srow, _K), :],
                                             idx_d2.at[b], isem[b])

                    def _wait_idx(b):
                        pltpu.make_async_copy(
                            src2.at[pl.ds(0, _K), :], idx_s2.at[b],
                            isem[b]).wait()
                        pltpu.make_async_copy(
                            src2.at[pl.ds(0, _K), :], idx_d2.at[b],
                            isem[b]).wait()

                    _fetch_idx(0, 0, True)

                    def ebody(i2, carry):
                        for m in range(2):
                            i = i2 * 2 + m

                            @pl.when(i < n_iters)
                            def _(i=i, m=m):
                                @pl.when(i + 1 < n_iters)
                                def _():
                                    _fetch_idx(i + 1, 1 - m, False)

                                @pl.when(i > 0)
                                def _():
                                    _wait_idx(m)

                                for kk in range(_K):
                                    def cbody(j, carry2, kk=kk):
                                        v = idx_d2[m, kk, pl.ds(j * 16, 16)]
                                        ok = (v >= lo) & (v < hi)
                                        loc = jnp.where(ok, v - lo,
                                                        R + (v & 127))
                                        idx_d2[m, kk, pl.ds(j * 16, 16)] = loc
                                        return carry2
                                    lax.fori_loop(0, _AB // 16, cbody, 0)

                                gd = [None] * _K
                                sd = [None] * _K
                                for kk in range(_K):
                                    if kk >= 4:
                                        sd[kk - 4].wait()
                                    gd[kk] = pltpu.async_copy(
                                        y_hbm.at[idx_s2.at[m, kk]],
                                        rows.at[kk % 4], gsem[kk % 4])
                                    if kk >= 1:
                                        gd[kk - 1].wait()
                                        sd[kk - 1] = pltpu.async_copy(
                                            rows.at[(kk - 1) % 4],
                                            acc_sp.at[idx_d2.at[m, kk - 1]],
                                            ssem[(kk - 1) % 4], add=True)
                                gd[_K - 1].wait()
                                sd[_K - 1] = pltpu.async_copy(
                                    rows.at[(_K - 1) % 4],
                                    acc_sp.at[idx_d2.at[m, _K - 1]],
                                    ssem[(_K - 1) % 4], add=True)
                                for kk in range(_K - 4, _K):
                                    sd[kk].wait()
                        return carry
                    lax.fori_loop(0, (n_iters + 1) // 2, ebody, 0)
                    plsc.subcore_barrier()

                    vfull = jnp.minimum(hi, n_dst) - lo
                    nfl = (fchunks + 15) // 16
                    nfl4 = (nfl + 3) // 4

                    def _full(j):
                        return (((j * 16 + t) * _AB + _AB <= vfull)
                                & (j >= 0) & (j < nfl))

                    def _wait_write(j, slot):
                        @pl.when(_full(j))
                        def _():
                            pltpu.make_async_copy(
                                rows.at[slot], outj.at[pl.ds(lo, _AB), :],
                                ssem[slot]).wait()

                    def fbody(j4, carry):
                        for m in range(4):
                            j = j4 * 4 + m
                            off = (j * 16 + t) * _AB
                            _wait_write(j - 4, m)

                            @pl.when(_full(j))
                            def _(off=off, m=m):
                                pltpu.sync_copy(
                                    acc_sp.at[pl.ds(off, _AB), :],
                                    rows.at[m])
                                pltpu.async_copy(
                                    rows.at[m],
                                    outj.at[pl.ds(lo + off, _AB), :],
                                    ssem[m])

                            @pl.when((off < vfull) & (off + _AB > vfull))
                            def _(off=off):
                                def sbody(mm, carry2):
                                    off2 = off + mm * 8

                                    @pl.when(off2 + 8 <= vfull)
                                    def _():
                                        pltpu.sync_copy(
                                            acc_sp.at[pl.ds(off2, 8), :],
                                            bounce8)
                                        pltpu.sync_copy(
                                            bounce8,
                                            outj.at[pl.ds(lo + off2, 8), :])
                                    return carry2
                                lax.fori_loop(0, _AB // 8, sbody, 0)
                        return carry
                    lax.fori_loop(0, nfl4, fbody, 0)
                    for m in range(4):
                        _wait_write((nfl4 - 1) * 4 + m, m)
                    plsc.subcore_barrier()
                return pcarry
            lax.fori_loop(0, npass, pbody, 0)

    return k


def _tc_xw(x, deg, W, bn=1000):
    n = x.shape[0]

    def body(x_ref, d_ref, w_ref, o_ref):
        nrm = lax.rsqrt(jnp.maximum(d_ref[...], 1.0))
        o_ref[...] = jnp.dot(x_ref[...] * nrm, w_ref[...],
                             preferred_element_type=jnp.float32)

    return pl.pallas_call(
        body,
        grid=(n // bn,),
        in_specs=[
            pl.BlockSpec((bn, x.shape[1]), lambda i: (i, 0)),
            pl.BlockSpec((bn, 1), lambda i: (i, 0)),
            pl.BlockSpec(W.shape, lambda i: (0, 0)),
        ],
        out_specs=pl.BlockSpec((bn, W.shape[1]), lambda i: (i, 0)),
        out_shape=jax.ShapeDtypeStruct((n, W.shape[1]), jnp.float32),
    )(x, deg, W)


def _tc_combine(accs, degs, biases, bn=1000):
    nrel = len(accs)
    n = accs[0].shape[0]

    def body(*refs):
        acc_refs = refs[:nrel]
        deg_refs = refs[nrel:2 * nrel]
        b_refs = refs[2 * nrel:3 * nrel]
        o_ref = refs[3 * nrel]
        tot = None
        for r in range(nrel):
            nrm = lax.rsqrt(jnp.maximum(deg_refs[r][...], 1.0))
            term = acc_refs[r][...] * nrm + b_refs[r][...]
            tot = term if tot is None else tot + term
        tot = tot * (1.0 / nrel)
        o_ref[...] = jnp.where(tot >= 0, tot, 0.01 * tot)

    in_specs = (
        [pl.BlockSpec((bn, _F), lambda i: (i, 0)) for _ in range(nrel)]
        + [pl.BlockSpec((bn, 1), lambda i: (i, 0)) for _ in range(nrel)]
        + [pl.BlockSpec((1, _F), lambda i: (0, 0)) for _ in range(nrel)]
    )
    return pl.pallas_call(
        body,
        grid=(n // bn,),
        in_specs=in_specs,
        out_specs=pl.BlockSpec((bn, _F), lambda i: (i, 0)),
        out_shape=jax.ShapeDtypeStruct((n, _F), jnp.float32),
    )(*accs, *degs, *[b.reshape(1, -1) for b in biases])


def _tc_final(h, W, b, bn=1000):
    n = h.shape[0]

    def body(h_ref, w_ref, b_ref, o_ref):
        o_ref[...] = jnp.dot(h_ref[...], w_ref[...],
                             preferred_element_type=jnp.float32) + b_ref[...]

    return pl.pallas_call(
        body,
        grid=(n // bn,),
        in_specs=[
            pl.BlockSpec((bn, _F), lambda i: (i, 0)),
            pl.BlockSpec(W.shape, lambda i: (0, 0)),
            pl.BlockSpec((1, W.shape[1]), lambda i: (0, 0)),
        ],
        out_specs=pl.BlockSpec((bn, W.shape[1]), lambda i: (i, 0)),
        out_shape=jax.ShapeDtypeStruct((n, W.shape[1]), jnp.float32),
    )(h, W, b.reshape(1, -1))


_DST_RELS = {
    "user": ["follows", "posted_by"],
    "news": ["posts", "publishes"],
    "source": ["published_by"],
}


def _pad_edges(src, dst, n_src, n_dst):
    ne = src.shape[0]
    ne_pad = ((ne + _STRIPE - 1) // _STRIPE) * _STRIPE
    pad = ne_pad - ne
    if pad:
        ar = jnp.arange(pad, dtype=jnp.int32)
        src = jnp.concatenate([src, (ar * 97) % n_src])
        dst = jnp.concatenate([dst, n_dst + (ar & 63)])
    return src.reshape(ne_pad // _AB, _AB), dst.reshape(ne_pad // _AB, _AB)


def kernel(x_user, x_news, x_source, follows_src, follows_dst, posted_by_src, posted_by_dst, posts_src, posts_dst, publishes_src, publishes_dst, published_by_src, published_by_dst, conv1_follows_W, conv1_follows_b, conv1_posted_by_W, conv1_posted_by_b, conv1_posts_W, conv1_posts_b, conv1_publishes_W, conv1_publishes_b, conv1_published_by_W, conv1_published_by_b, conv2_follows_W, conv2_follows_b, conv2_posted_by_W, conv2_posted_by_b, conv2_posts_W, conv2_posts_b, conv2_publishes_W, conv2_publishes_b, conv2_published_by_W, conv2_published_by_b, lin_user_W, lin_user_b, lin_news_W, lin_news_b, lin_source_W, lin_source_b):
    d = dict(locals())
    edge_idx = []
    for name, st, dt, ne in _RELS:
        edge_idx.append(d[f"{name}_src"])
        edge_idx.append(d[f"{name}_dst"])
    degs = _degrees(edge_idx)
    deg_out = {}
    deg_in = {}
    for j, (name, st, dt, ne) in enumerate(_RELS):
        deg_out[name] = degs[2 * j].reshape(-1, 1)
        deg_in[name] = degs[2 * j + 1].reshape(-1, 1)

    padded = {}
    for name, st, dt, ne in _RELS:
        padded[name] = _pad_edges(d[f"{name}_src"], d[f"{name}_dst"],
                                  _NNODES[st], _NNODES[dt])

    xs = {"user": x_user, "news": x_news, "source": x_source}
    zeros_in = jnp.zeros((16, _F), jnp.float32)

    def hetero(xs_in, layer):
        ys = {}
        for name, st, dt, ne in _RELS:
            ys[name] = _tc_xw(xs_in[st], deg_out[name],
                              d[f"{layer}_{name}_W"])
        out = {}
        for dt_name, R, npass in _AGG_PLAN:
            n_dst = _NNODES[dt_name]
            rels = _DST_RELS[dt_name]
            rel_plan = []
            agg_in = []
            for rn in rels:
                s2, d2 = padded[rn]
                rel_plan.append((s2.shape[0] * _AB, _REL_BY_NAME[rn][3]))
                agg_in += [s2, d2, ys[rn]]
            agg_in.append(zeros_in)
            accs = _sc_agg_kernel(n_dst, R, npass, rel_plan)(*agg_in)
            if not isinstance(accs, (list, tuple)):
                accs = [accs]
            out[dt_name] = _tc_combine(
                list(accs),
                [deg_in[rn] for rn in rels],
                [d[f"{layer}_{rn}_b"] for rn in rels])
        return out

    h = hetero(xs, "conv1")
    h1 = hetero(h, "conv2")
    hout = {
        "user": _tc_final(h1["user"], lin_user_W, lin_user_b),
        "news": _tc_final(h1["news"], lin_news_W, lin_news_b),
        "source": _tc_final(h1["source"], lin_source_W, lin_source_b),
    }
    return (hout["user"], hout["news"], hout["source"],
            h1["user"], h1["news"], h1["source"])
